# baseline (speedup 1.0000x reference)
import functools

import jax
import jax.numpy as jnp
from jax.experimental import pallas as pl
from jax.experimental.pallas import tpu as pltpu

B, S, D, DS = 4, 2048, 1024, 64
NF, NR, NT = 64, 32, 48
N_ALL = NF + NR + NT
TKF, TKR, TKT = 8, 4, 6


def _topk_mask_normalize(w, k, n):
    wt = jnp.swapaxes(w, 0, 1)
    il = jax.lax.broadcasted_iota(jnp.int32, (1, n), 1)
    jt = jax.lax.broadcasted_iota(jnp.int32, (n, 1), 0)
    beats = (wt > w) | ((wt == w) & (jt < il))
    rank = jnp.sum(beats.astype(jnp.float32), axis=0, keepdims=True)
    sparse = jnp.where(rank < k, w, 0.0)
    return sparse / (jnp.sum(sparse, axis=1, keepdims=True) + 1e-8)


BLK = 512
NS = S // BLK


def _router_kernel(x_ref, imp_ref, w_ref, b_ref, ent_ref,
                   of_ref, or_ref, ot_ref, p_buf):
    s = pl.program_id(1)

    h = jnp.dot(x_ref[0], w_ref[...], preferred_element_type=jnp.float32)
    h = h + b_ref[...]
    al = jnp.dot(h, ent_ref[...],
                 preferred_element_type=jnp.float32)

    def seg(lo, n):
        z = al[:, lo:lo + n]
        m = jnp.max(z, axis=1, keepdims=True)
        e = jnp.exp(z - m)
        return e / jnp.sum(e, axis=1, keepdims=True)

    p = jnp.concatenate([seg(0, NF), seg(NF, NR), seg(NF + NR, NT)],
                        axis=1)
    p_buf[pl.ds(s * BLK, BLK), :] = p

    @pl.when(s == NS - 1)
    def _():
        pooled = jnp.dot(imp_ref[0], p_buf[...],
                         preferred_element_type=jnp.float32)
        of_ref[0] = _topk_mask_normalize(pooled[:, :NF], TKF, NF)
        or_ref[0] = _topk_mask_normalize(pooled[:, NF:NF + NR], TKR, NR)
        ot_ref[0] = _topk_mask_normalize(pooled[:, NF + NR:], TKT, NT)


@functools.partial(jax.jit, static_argnames=("interpret",))
def kernel(x, importance, W_proj, b_proj, neuron_emb, interpret=False):
    imp3 = importance.reshape(B, 1, S)
    b2 = b_proj.reshape(1, DS)
    emb_norm = neuron_emb / (jnp.linalg.norm(neuron_emb, axis=-1,
                                             keepdims=True) + 1e-12)
    ent = emb_norm.T

    of, orr, ot = pl.pallas_call(
        _router_kernel,
        grid=(B, NS),
        in_specs=[
            pl.BlockSpec((1, BLK, D), lambda b, s: (b, s, 0)),
            pl.BlockSpec((1, 1, S), lambda b, s: (b, 0, 0)),
            pl.BlockSpec((D, DS), lambda b, s: (0, 0)),
            pl.BlockSpec((1, DS), lambda b, s: (0, 0)),
            pl.BlockSpec((DS, N_ALL), lambda b, s: (0, 0)),
        ],
        out_specs=[
            pl.BlockSpec((1, 1, NF), lambda b, s: (b, 0, 0)),
            pl.BlockSpec((1, 1, NR), lambda b, s: (b, 0, 0)),
            pl.BlockSpec((1, 1, NT), lambda b, s: (b, 0, 0)),
        ],
        out_shape=[
            jax.ShapeDtypeStruct((B, 1, NF), jnp.float32),
            jax.ShapeDtypeStruct((B, 1, NR), jnp.float32),
            jax.ShapeDtypeStruct((B, 1, NT), jnp.float32),
        ],
        scratch_shapes=[
            pltpu.VMEM((S, N_ALL), jnp.float32),
        ],
        compiler_params=pltpu.CompilerParams(
            dimension_semantics=("parallel", "arbitrary"),
        ),
        interpret=interpret,
    )(x, imp3, W_proj, b2, ent)

    of, orr, ot = of[:, 0], orr[:, 0], ot[:, 0]
    return jnp.concatenate([of, orr, orr, ot], axis=-1)


# speedup vs baseline: 1.6552x; 1.6552x over previous
import functools

import jax
import jax.numpy as jnp
from jax.experimental import pallas as pl
from jax.experimental.pallas import tpu as pltpu

B, S, D, DS = 4, 2048, 1024, 64
NF, NR, NT = 64, 32, 48
N_ALL = NF + NR + NT
TKF, TKR, TKT = 8, 4, 6

SEGW = 128
N_PAD = 3 * SEGW
OF_F, OF_R, OF_T = 0, SEGW, 2 * SEGW

BLK = 512
NS = S // BLK


def _topk_mask_normalize(w, k, n):
    wt = jnp.swapaxes(w, 0, 1)
    il = jax.lax.broadcasted_iota(jnp.int32, (1, n), 1)
    jt = jax.lax.broadcasted_iota(jnp.int32, (n, 1), 0)
    beats = (wt > w) | ((wt == w) & (jt < il))
    rank = jnp.sum(beats.astype(jnp.float32), axis=0, keepdims=True)
    sparse = jnp.where(rank < k, w, 0.0)
    return sparse / (jnp.sum(sparse, axis=1, keepdims=True) + 1e-8)


def _router_kernel(x_ref, imp_ref, w_ref, b_ref, ent_ref,
                   of_ref, or_ref, ot_ref, p_buf):
    s = pl.program_id(1)

    h = jnp.dot(x_ref[0], w_ref[...], preferred_element_type=jnp.float32)
    h = h + b_ref[...]
    al = jnp.dot(h, ent_ref[...],
                 preferred_element_type=jnp.float32)

    lane = jax.lax.broadcasted_iota(jnp.int32, (BLK, N_PAD), 1)

    def bc3(vf, vr, vt):
        return jnp.where(lane < SEGW, vf,
                         jnp.where(lane < 2 * SEGW, vr, vt))

    m_bc = bc3(jnp.max(al[:, OF_F:OF_F + NF], axis=1, keepdims=True),
               jnp.max(al[:, OF_R:OF_R + NR], axis=1, keepdims=True),
               jnp.max(al[:, OF_T:OF_T + NT], axis=1, keepdims=True))
    e = jnp.exp(al - m_bc)
    s_bc = bc3(jnp.sum(e[:, OF_F:OF_F + NF], axis=1, keepdims=True),
               jnp.sum(e[:, OF_R:OF_R + NR], axis=1, keepdims=True),
               jnp.sum(e[:, OF_T:OF_T + NT], axis=1, keepdims=True))
    p_buf[pl.ds(s * BLK, BLK), :] = e / s_bc

    @pl.when(s == NS - 1)
    def _():
        pooled = jnp.dot(imp_ref[0], p_buf[...],
                         preferred_element_type=jnp.float32)
        of_ref[0] = _topk_mask_normalize(pooled[:, OF_F:OF_F + NF], TKF, NF)
        or_ref[0] = _topk_mask_normalize(pooled[:, OF_R:OF_R + NR], TKR, NR)
        ot_ref[0] = _topk_mask_normalize(pooled[:, OF_T:OF_T + NT], TKT, NT)


@functools.partial(jax.jit, static_argnames=("interpret",))
def kernel(x, importance, W_proj, b_proj, neuron_emb, interpret=False):
    imp3 = importance.reshape(B, 1, S)
    b2 = b_proj.reshape(1, DS)
    emb_norm = neuron_emb / (jnp.linalg.norm(neuron_emb, axis=-1,
                                             keepdims=True) + 1e-12)
    ent = emb_norm.T
    zf = jnp.zeros((DS, SEGW - NF), dtype=jnp.float32)
    zr = jnp.zeros((DS, SEGW - NR), dtype=jnp.float32)
    zt = jnp.zeros((DS, SEGW - NT), dtype=jnp.float32)
    ent_pad = jnp.concatenate(
        [ent[:, :NF], zf, ent[:, NF:NF + NR], zr, ent[:, NF + NR:], zt],
        axis=1)

    of, orr, ot = pl.pallas_call(
        _router_kernel,
        grid=(B, NS),
        in_specs=[
            pl.BlockSpec((1, BLK, D), lambda b, s: (b, s, 0)),
            pl.BlockSpec((1, 1, S), lambda b, s: (b, 0, 0)),
            pl.BlockSpec((D, DS), lambda b, s: (0, 0)),
            pl.BlockSpec((1, DS), lambda b, s: (0, 0)),
            pl.BlockSpec((DS, N_PAD), lambda b, s: (0, 0)),
        ],
        out_specs=[
            pl.BlockSpec((1, 1, NF), lambda b, s: (b, 0, 0)),
            pl.BlockSpec((1, 1, NR), lambda b, s: (b, 0, 0)),
            pl.BlockSpec((1, 1, NT), lambda b, s: (b, 0, 0)),
        ],
        out_shape=[
            jax.ShapeDtypeStruct((B, 1, NF), jnp.float32),
            jax.ShapeDtypeStruct((B, 1, NR), jnp.float32),
            jax.ShapeDtypeStruct((B, 1, NT), jnp.float32),
        ],
        scratch_shapes=[
            pltpu.VMEM((S, N_PAD), jnp.float32),
        ],
        compiler_params=pltpu.CompilerParams(
            dimension_semantics=("parallel", "arbitrary"),
        ),
        interpret=interpret,
    )(x, imp3, W_proj, b2, ent_pad)

    of, orr, ot = of[:, 0], orr[:, 0], ot[:, 0]
    return jnp.concatenate([of, orr, orr, ot], axis=-1)
